# Initial kernel scaffold; baseline (speedup 1.0000x reference)
import jax
import jax.numpy as jnp
from jax.experimental import pallas as pl


def kernel(f0, f1, item_emb, W, b, alphas, edge_index, k):
    raise NotImplementedError("write your pallas kernel here")


# speedup vs baseline: 12.0018x; 12.0018x over previous
import jax
import jax.numpy as jnp
from jax import lax
from jax.experimental import pallas as pl
from jax.experimental.pallas import tpu as pltpu
from jax.experimental.pallas import tpu_sc as plsc

N = 10000
E = 320000
DM = 64
EMB = 128
F2 = 2 * DM

NC = 2
NS = 16
L = 16
NW = NC * NS
EW = E // NW
C = 80
NCHUNK = EW // C
NG = C // L
NR = (N + 127) // 128
NP = NR * 128
SROWS = NP // NS

EPS = 1e-8

_mesh = plsc.VectorSubcoreMesh(core_axis_name="c", subcore_axis_name="s")
_sc_params = pltpu.CompilerParams(needs_layout_passes=False)


def _sigmoid(x):
    return 1.0 / (1.0 + jnp.exp(-x))


def _rc(idx):
    return lax.shift_right_logical(idx, 7), jnp.bitwise_and(idx, 127)


def _sc1_body(f01, srcr, dstr, nrm, idrows,
              s0o, s1o, inwo,
              idxs_v, idxd_v, rows_s, rows_d, n0_v, n1_v,
              sb0_v, sb1_v, inw0_v, inw1_v, idr_v,
              inw0_sh, inw1_sh):
    cid = lax.axis_index("c")
    sid = lax.axis_index("s")
    wid = sid * NC + cid
    base = wid * EW

    pltpu.sync_copy(nrm.at[0], n0_v)
    pltpu.sync_copy(nrm.at[1], n1_v)
    pltpu.sync_copy(idrows, idr_v)

    z16 = jnp.zeros((L,), jnp.float32)

    def zacc(j, carry):
        for cc in range(128 // L):
            inw0_v[j, pl.ds(cc * L, L)] = z16
            inw1_v[j, pl.ds(cc * L, L)] = z16
        return carry

    lax.fori_loop(0, NR, zacc, 0)

    @pl.when(sid == 0)
    def _():
        pltpu.sync_copy(inw0_v, inw0_sh)
        pltpu.sync_copy(inw1_v, inw1_sh)

    lane = lax.iota(jnp.int32, L)

    def chunk(i, carry):
        off = base + i * C
        pltpu.sync_copy(srcr.at[pl.ds(off, C)], idxs_v)
        pltpu.sync_copy(dstr.at[pl.ds(off, C)], idxd_v)
        pltpu.sync_copy(f01.at[idxs_v], rows_s)
        pltpu.sync_copy(f01.at[idxd_v], rows_d)

        def group(g, gcarry):
            def edge(j, ecarry):
                d0v, d1v = ecarry
                e = g * L + j
                a0 = rows_s[e, pl.ds(0, L)] * rows_d[e, pl.ds(0, L)]
                a0 = a0 + rows_s[e, pl.ds(L, L)] * rows_d[e, pl.ds(L, L)]
                a0 = a0 + rows_s[e, pl.ds(2 * L, L)] * rows_d[e, pl.ds(2 * L, L)]
                a0 = a0 + rows_s[e, pl.ds(3 * L, L)] * rows_d[e, pl.ds(3 * L, L)]
                a1 = rows_s[e, pl.ds(4 * L, L)] * rows_d[e, pl.ds(4 * L, L)]
                a1 = a1 + rows_s[e, pl.ds(5 * L, L)] * rows_d[e, pl.ds(5 * L, L)]
                a1 = a1 + rows_s[e, pl.ds(6 * L, L)] * rows_d[e, pl.ds(6 * L, L)]
                a1 = a1 + rows_s[e, pl.ds(7 * L, L)] * rows_d[e, pl.ds(7 * L, L)]
                m = lane == j
                d0v = jnp.where(m, jnp.sum(a0), d0v)
                d1v = jnp.where(m, jnp.sum(a1), d1v)
                return d0v, d1v

            z = jnp.zeros((L,), jnp.float32)
            d0, d1 = lax.fori_loop(0, L, edge, (z, z))

            i16s = idxs_v[pl.ds(g * L, L)]
            i16d = idxd_v[pl.ds(g * L, L)]
            rs, cs = _rc(i16s)
            rd, cd = _rc(i16d)
            nu0 = plsc.load_gather(n0_v, [rs, cs])
            nv0 = plsc.load_gather(n0_v, [rd, cd])
            nu1 = plsc.load_gather(n1_v, [rs, cs])
            nv1 = plsc.load_gather(n1_v, [rd, cd])
            sc0 = _sigmoid(_sigmoid(d0) / (nu0 * nv0 + EPS))
            sc1 = _sigmoid(_sigmoid(d1) / (nu1 * nv1 + EPS))
            sb0_v[pl.ds(g * L, L)] = sc0
            sb1_v[pl.ds(g * L, L)] = sc1
            plsc.addupdate_scatter(inw0_v, [rd, cd], sc0)
            plsc.addupdate_scatter(inw1_v, [rd, cd], sc1)
            return gcarry

        lax.fori_loop(0, NG, group, 0)

        pltpu.sync_copy(sb0_v, s0o.at[pl.ds(off, C)])
        pltpu.sync_copy(sb1_v, s1o.at[pl.ds(off, C)])
        return carry

    lax.fori_loop(0, NCHUNK, chunk, 0)

    plsc.subcore_barrier()
    pltpu.sync_copy(inw0_v, inw0_sh.at[idr_v], add=True)
    pltpu.sync_copy(inw1_v, inw1_sh.at[idr_v], add=True)
    plsc.subcore_barrier()

    @pl.when(sid == 0)
    def _():
        pltpu.sync_copy(inw0_sh, inwo.at[cid, 0])
        pltpu.sync_copy(inw1_sh, inwo.at[cid, 1])


_sc1 = pl.kernel(
    _sc1_body,
    out_type=[
        jax.ShapeDtypeStruct((E,), jnp.float32),
        jax.ShapeDtypeStruct((E,), jnp.float32),
        jax.ShapeDtypeStruct((NC, 2, NR, 128), jnp.float32),
    ],
    mesh=_mesh,
    compiler_params=_sc_params,
    scratch_types=[
        pltpu.VMEM((C,), jnp.int32),
        pltpu.VMEM((C,), jnp.int32),
        pltpu.VMEM((C, F2), jnp.float32),
        pltpu.VMEM((C, F2), jnp.float32),
        pltpu.VMEM((NR, 128), jnp.float32),
        pltpu.VMEM((NR, 128), jnp.float32),
        pltpu.VMEM((C,), jnp.float32),
        pltpu.VMEM((C,), jnp.float32),
        pltpu.VMEM((NR, 128), jnp.float32),
        pltpu.VMEM((NR, 128), jnp.float32),
        pltpu.VMEM((NR,), jnp.int32),
        pltpu.VMEM_SHARED((NR, 128), jnp.float32),
        pltpu.VMEM_SHARED((NR, 128), jnp.float32),
    ],
)


def _sc2_body(h, srcr, dstr, s0i, s1i, rn,
              outo,
              idxs_v, idxd_v, s0b_v, s1b_v, wb_v, rows_v, rn0_v, rn1_v, zb_v,
              out_sh):
    cid = lax.axis_index("c")
    sid = lax.axis_index("s")
    wid = sid * NC + cid
    base = wid * EW

    pltpu.sync_copy(rn.at[0], rn0_v)
    pltpu.sync_copy(rn.at[1], rn1_v)

    z16 = jnp.zeros((L,), jnp.float32)

    ZROWS = 104

    def zzb(j, carry):
        for cc in range(EMB // L):
            zb_v[j, pl.ds(cc * L, L)] = z16
        return carry

    lax.fori_loop(0, ZROWS, zzb, 0)
    myrow = sid * SROWS
    for p in range(6):
        pltpu.sync_copy(zb_v, out_sh.at[pl.ds(myrow + p * ZROWS, ZROWS)])
    pltpu.sync_copy(zb_v.at[pl.ds(0, 8)],
                    out_sh.at[pl.ds(myrow + 6 * ZROWS, 8)])
    plsc.subcore_barrier()

    def chunk(i, carry):
        off = base + i * C
        pltpu.sync_copy(srcr.at[pl.ds(off, C)], idxs_v)
        pltpu.sync_copy(dstr.at[pl.ds(off, C)], idxd_v)
        pltpu.sync_copy(s0i.at[pl.ds(off, C)], s0b_v)
        pltpu.sync_copy(s1i.at[pl.ds(off, C)], s1b_v)
        pltpu.sync_copy(h.at[idxs_v], rows_v)

        def group(g, gcarry):
            i16s = idxs_v[pl.ds(g * L, L)]
            i16d = idxd_v[pl.ds(g * L, L)]
            rs, cs = _rc(i16s)
            rd, cd = _rc(i16d)
            w = (s0b_v[pl.ds(g * L, L)]
                 * plsc.load_gather(rn0_v, [rs, cs])
                 * plsc.load_gather(rn0_v, [rd, cd])
                 + s1b_v[pl.ds(g * L, L)]
                 * plsc.load_gather(rn1_v, [rs, cs])
                 * plsc.load_gather(rn1_v, [rd, cd]))
            wb_v[pl.ds(g * L, L)] = w
            return gcarry

        lax.fori_loop(0, NG, group, 0)

        def edge(j, ecarry):
            wj = wb_v[pl.ds(j, L)][0]
            for cc in range(EMB // L):
                rows_v[j, pl.ds(cc * L, L)] = rows_v[j, pl.ds(cc * L, L)] * wj
            return ecarry

        lax.fori_loop(0, C, edge, 0)

        pltpu.sync_copy(rows_v, out_sh.at[idxd_v], add=True)
        return carry

    lax.fori_loop(0, NCHUNK, chunk, 0)

    plsc.subcore_barrier()
    pltpu.sync_copy(out_sh.at[pl.ds(myrow, SROWS)],
                    outo.at[cid, pl.ds(myrow, SROWS)])


_sc2 = pl.kernel(
    _sc2_body,
    out_type=jax.ShapeDtypeStruct((NC, NP, EMB), jnp.float32),
    mesh=_mesh,
    compiler_params=_sc_params,
    scratch_types=[
        pltpu.VMEM((C,), jnp.int32),
        pltpu.VMEM((C,), jnp.int32),
        pltpu.VMEM((C,), jnp.float32),
        pltpu.VMEM((C,), jnp.float32),
        pltpu.VMEM((C + L,), jnp.float32),
        pltpu.VMEM((C, EMB), jnp.float32),
        pltpu.VMEM((NR, 128), jnp.float32),
        pltpu.VMEM((NR, 128), jnp.float32),
        pltpu.VMEM((104, EMB), jnp.float32),
        pltpu.VMEM_SHARED((NP, EMB), jnp.float32),
    ],
)


def _tca_body(f01_ref, wt_ref, b_ref, emb_ref, av_ref, h_ref, nmat_ref):
    x = f01_ref[...] * av_ref[...]
    h_ref[...] = (jnp.dot(x, wt_ref[...], preferred_element_type=jnp.float32)
                  + b_ref[...] + emb_ref[...])
    sq = f01_ref[...] * f01_ref[...]
    n0 = jnp.sqrt(jnp.sum(sq[:, :DM], axis=1, keepdims=True))
    n1 = jnp.sqrt(jnp.sum(sq[:, DM:], axis=1, keepdims=True))
    nmat_ref[...] = jnp.concatenate([n0, n1], axis=1)


def _tcb_body(inwp_ref, carr_ref, rn_ref):
    t = inwp_ref[0] + inwp_ref[1] + EPS
    rn_ref[...] = lax.rsqrt(t) * carr_ref[:, 0:1]


def _tcc_body(p_ref, o_ref):
    o_ref[...] = p_ref[0, :N, :] + p_ref[1, :N, :]


def kernel(f0, f1, item_emb, W, b, alphas, edge_index, k):
    alph = jax.nn.softmax(alphas, axis=0)
    a0 = alph[0]
    a1 = alph[1]

    f01 = jnp.concatenate([f0, f1], axis=1)
    src = edge_index[0]
    dst = edge_index[1]
    wt = W.T
    b2 = b.reshape(1, EMB)
    avec = jnp.concatenate([jnp.full((1, DM), a0, jnp.float32),
                            jnp.full((1, DM), a1, jnp.float32)], axis=1)

    h, nmat = pl.pallas_call(
        _tca_body,
        out_shape=[
            jax.ShapeDtypeStruct((N, EMB), jnp.float32),
            jax.ShapeDtypeStruct((N, 2), jnp.float32),
        ],
    )(f01, wt, b2, item_emb, avec)

    nrm = jnp.pad(nmat.T, ((0, 0), (0, NP - N))).reshape(2, NR, 128)

    idrows = jnp.arange(NR, dtype=jnp.int32)

    s0, s1, inwp = _sc1(f01, src, dst, nrm, idrows)

    inw2 = inwp.reshape(NC, 2, NP)
    kf = jnp.float32(k)
    carr = jnp.tile(jnp.stack([jnp.sqrt(a0 / kf), jnp.sqrt(a1 / kf)])[:, None],
                    (1, EMB))

    rn = pl.pallas_call(
        _tcb_body,
        out_shape=jax.ShapeDtypeStruct((2, NP), jnp.float32),
    )(inw2, carr)

    outp = _sc2(h, src, dst, s0, s1, rn.reshape(2, NR, 128))

    out = pl.pallas_call(
        _tcc_body,
        out_shape=jax.ShapeDtypeStruct((N, EMB), jnp.float32),
    )(outp)
    return out


# speedup vs baseline: 16.9868x; 1.4154x over previous
import jax
import jax.numpy as jnp
from jax import lax
from jax.experimental import pallas as pl
from jax.experimental.pallas import tpu as pltpu
from jax.experimental.pallas import tpu_sc as plsc

N = 10000
E = 320000
DM = 64
EMB = 128
F2 = 2 * DM

NC = 2
NS = 16
L = 16
NW = NC * NS
EW = E // NW
CF = 128
NF = EW // CF
NPAIR = NF // 2
TAIL = EW - NF * CF
TOFF = NF * CF
NGF = CF // L
NR = (N + 127) // 128
NP = NR * 128
SROWS = NP // NS

EPS = 1e-8

_mesh = plsc.VectorSubcoreMesh(core_axis_name="c", subcore_axis_name="s")
_sc_params = pltpu.CompilerParams(needs_layout_passes=False)


def _sigmoid(x):
    return 1.0 / (1.0 + jnp.exp(-x))


def _rc(idx):
    return lax.shift_right_logical(idx, 7), jnp.bitwise_and(idx, 127)


def _sc1_body(f01, srcr, dstr, nrm, idrows,
              s0o, s1o, inwo,
              ibs_a, ibd_a, ibs_b, ibd_b,
              rows_sa, rows_da, rows_sb, rows_db,
              n0_v, n1_v, inw0_v, inw1_v, sb_v, idr_v, tidx_v,
              inw0_sh, inw1_sh,
              sem_ga, sem_gb, sem_ia, sem_ib, sem_w):
    cid = lax.axis_index("c")
    sid = lax.axis_index("s")
    wid = sid * NC + cid
    base = wid * EW

    pltpu.sync_copy(nrm.at[0], n0_v)
    pltpu.sync_copy(nrm.at[1], n1_v)
    pltpu.sync_copy(idrows, idr_v)

    z16 = jnp.zeros((L,), jnp.float32)

    def zacc(j, carry):
        for cc in range(128 // L):
            inw0_v[j, pl.ds(cc * L, L)] = z16
            inw1_v[j, pl.ds(cc * L, L)] = z16
        return carry

    lax.fori_loop(0, NR, zacc, 0)

    @pl.when(sid == 0)
    def _():
        pltpu.sync_copy(inw0_v, inw0_sh)
        pltpu.sync_copy(inw1_v, inw1_sh)

    lane = lax.iota(jnp.int32, L)

    def compute_chunk(ibs, ibd, rows_s, rows_d, sbrow):
        def group(g, gcarry):
            def edge(j, ecarry):
                d0v, d1v = ecarry
                e = g * L + j
                a0 = rows_s[e, pl.ds(0, L)] * rows_d[e, pl.ds(0, L)]
                a0 = a0 + rows_s[e, pl.ds(L, L)] * rows_d[e, pl.ds(L, L)]
                a0 = a0 + rows_s[e, pl.ds(2 * L, L)] * rows_d[e, pl.ds(2 * L, L)]
                a0 = a0 + rows_s[e, pl.ds(3 * L, L)] * rows_d[e, pl.ds(3 * L, L)]
                a1 = rows_s[e, pl.ds(4 * L, L)] * rows_d[e, pl.ds(4 * L, L)]
                a1 = a1 + rows_s[e, pl.ds(5 * L, L)] * rows_d[e, pl.ds(5 * L, L)]
                a1 = a1 + rows_s[e, pl.ds(6 * L, L)] * rows_d[e, pl.ds(6 * L, L)]
                a1 = a1 + rows_s[e, pl.ds(7 * L, L)] * rows_d[e, pl.ds(7 * L, L)]
                m = lane == j
                d0v = jnp.where(m, jnp.sum(a0), d0v)
                d1v = jnp.where(m, jnp.sum(a1), d1v)
                return d0v, d1v

            z = jnp.zeros((L,), jnp.float32)
            d0, d1 = lax.fori_loop(0, L, edge, (z, z))

            i16s = ibs[pl.ds(g * L, L)]
            i16d = ibd[pl.ds(g * L, L)]
            rs, cs = _rc(i16s)
            rd, cd = _rc(i16d)
            nu0 = plsc.load_gather(n0_v, [rs, cs])
            nv0 = plsc.load_gather(n0_v, [rd, cd])
            nu1 = plsc.load_gather(n1_v, [rs, cs])
            nv1 = plsc.load_gather(n1_v, [rd, cd])
            sc0 = _sigmoid(_sigmoid(d0) / (nu0 * nv0 + EPS))
            sc1 = _sigmoid(_sigmoid(d1) / (nu1 * nv1 + EPS))
            sb_v[sbrow, pl.ds(g * L, L)] = sc0
            sb_v[sbrow + 1, pl.ds(g * L, L)] = sc1
            plsc.addupdate_scatter(inw0_v, [rd, cd], sc0)
            plsc.addupdate_scatter(inw1_v, [rd, cd], sc1)
            return gcarry

        lax.fori_loop(0, NGF, group, 0)

    def start_idx(i, ibs, ibd, sem):
        off = base + i * CF
        pltpu.async_copy(srcr.at[pl.ds(off, CF)], ibs, sem)
        pltpu.async_copy(dstr.at[pl.ds(off, CF)], ibd, sem)

    def wait_idx(ibs, ibd, sem):
        pltpu.make_async_copy(srcr.at[pl.ds(base, CF)], ibs, sem).wait()
        pltpu.make_async_copy(dstr.at[pl.ds(base, CF)], ibd, sem).wait()

    def start_gather(ibs, ibd, rows_s, rows_d, sem):
        pltpu.async_copy(f01.at[ibs], rows_s, sem)
        pltpu.async_copy(f01.at[ibd], rows_d, sem)

    def wait_gather(ibs, ibd, rows_s, rows_d, sem):
        pltpu.make_async_copy(f01.at[ibs], rows_s, sem).wait()
        pltpu.make_async_copy(f01.at[ibd], rows_d, sem).wait()

    def start_scores(i, sbrow):
        off = base + i * CF
        pltpu.async_copy(sb_v.at[sbrow], s0o.at[pl.ds(off, CF)], sem_w)
        pltpu.async_copy(sb_v.at[sbrow + 1], s1o.at[pl.ds(off, CF)], sem_w)

    def drain_scores():
        pltpu.make_async_copy(sb_v.at[0], s0o.at[pl.ds(base, CF)], sem_w).wait()
        pltpu.make_async_copy(sb_v.at[1], s1o.at[pl.ds(base, CF)], sem_w).wait()

    start_idx(0, ibs_a, ibd_a, sem_ia)
    wait_idx(ibs_a, ibd_a, sem_ia)
    start_gather(ibs_a, ibd_a, rows_sa, rows_da, sem_ga)
    start_idx(1, ibs_b, ibd_b, sem_ib)
    wait_idx(ibs_b, ibd_b, sem_ib)
    start_gather(ibs_b, ibd_b, rows_sb, rows_db, sem_gb)

    def pair(p, carry):
        wait_gather(ibs_a, ibd_a, rows_sa, rows_da, sem_ga)

        @pl.when(p > 0)
        def _():
            drain_scores()
        compute_chunk(ibs_a, ibd_a, rows_sa, rows_da, 0)
        start_scores(2 * p, 0)

        @pl.when(p < NPAIR - 1)
        def _():
            start_idx(2 * p + 2, ibs_a, ibd_a, sem_ia)
            wait_idx(ibs_a, ibd_a, sem_ia)
            start_gather(ibs_a, ibd_a, rows_sa, rows_da, sem_ga)

        wait_gather(ibs_b, ibd_b, rows_sb, rows_db, sem_gb)

        @pl.when(p > 0)
        def _():
            drain_scores()
        compute_chunk(ibs_b, ibd_b, rows_sb, rows_db, 2)
        start_scores(2 * p + 1, 2)

        @pl.when(p < NPAIR - 1)
        def _():
            start_idx(2 * p + 3, ibs_b, ibd_b, sem_ib)
            wait_idx(ibs_b, ibd_b, sem_ib)
            start_gather(ibs_b, ibd_b, rows_sb, rows_db, sem_gb)
        return carry

    lax.fori_loop(0, NPAIR, pair, 0)
    drain_scores()
    drain_scores()

    toff = base + TOFF
    pltpu.sync_copy(srcr.at[pl.ds(toff, TAIL)], tidx_v.at[0])
    pltpu.sync_copy(dstr.at[pl.ds(toff, TAIL)], tidx_v.at[1])
    pltpu.sync_copy(f01.at[tidx_v.at[0]], rows_sa.at[pl.ds(0, TAIL)])
    pltpu.sync_copy(f01.at[tidx_v.at[1]], rows_da.at[pl.ds(0, TAIL)])

    def tedge(j, ecarry):
        d0v, d1v = ecarry
        a0 = rows_sa[j, pl.ds(0, L)] * rows_da[j, pl.ds(0, L)]
        a0 = a0 + rows_sa[j, pl.ds(L, L)] * rows_da[j, pl.ds(L, L)]
        a0 = a0 + rows_sa[j, pl.ds(2 * L, L)] * rows_da[j, pl.ds(2 * L, L)]
        a0 = a0 + rows_sa[j, pl.ds(3 * L, L)] * rows_da[j, pl.ds(3 * L, L)]
        a1 = rows_sa[j, pl.ds(4 * L, L)] * rows_da[j, pl.ds(4 * L, L)]
        a1 = a1 + rows_sa[j, pl.ds(5 * L, L)] * rows_da[j, pl.ds(5 * L, L)]
        a1 = a1 + rows_sa[j, pl.ds(6 * L, L)] * rows_da[j, pl.ds(6 * L, L)]
        a1 = a1 + rows_sa[j, pl.ds(7 * L, L)] * rows_da[j, pl.ds(7 * L, L)]
        m = lane == j
        d0v = jnp.where(m, jnp.sum(a0), d0v)
        d1v = jnp.where(m, jnp.sum(a1), d1v)
        return d0v, d1v

    zt = jnp.zeros((L,), jnp.float32)
    d0, d1 = lax.fori_loop(0, TAIL, tedge, (zt, zt))
    i16s = tidx_v[0, pl.ds(0, L)]
    i16d = tidx_v[1, pl.ds(0, L)]
    rs, cs = _rc(i16s)
    rd, cd = _rc(i16d)
    nu0 = plsc.load_gather(n0_v, [rs, cs])
    nv0 = plsc.load_gather(n0_v, [rd, cd])
    nu1 = plsc.load_gather(n1_v, [rs, cs])
    nv1 = plsc.load_gather(n1_v, [rd, cd])
    sc0 = _sigmoid(_sigmoid(d0) / (nu0 * nv0 + EPS))
    sc1 = _sigmoid(_sigmoid(d1) / (nu1 * nv1 + EPS))
    sb_v[0, pl.ds(0, L)] = sc0
    sb_v[1, pl.ds(0, L)] = sc1
    plsc.addupdate_scatter(inw0_v, [rd, cd], sc0)
    plsc.addupdate_scatter(inw1_v, [rd, cd], sc1)
    pltpu.sync_copy(sb_v.at[0, pl.ds(0, TAIL)], s0o.at[pl.ds(toff, TAIL)])
    pltpu.sync_copy(sb_v.at[1, pl.ds(0, TAIL)], s1o.at[pl.ds(toff, TAIL)])

    plsc.subcore_barrier()
    pltpu.sync_copy(inw0_v, inw0_sh.at[idr_v], add=True)
    pltpu.sync_copy(inw1_v, inw1_sh.at[idr_v], add=True)
    plsc.subcore_barrier()

    @pl.when(sid == 0)
    def _():
        pltpu.sync_copy(inw0_sh, inwo.at[cid, 0])
        pltpu.sync_copy(inw1_sh, inwo.at[cid, 1])


_sc1 = pl.kernel(
    _sc1_body,
    out_type=[
        jax.ShapeDtypeStruct((E,), jnp.float32),
        jax.ShapeDtypeStruct((E,), jnp.float32),
        jax.ShapeDtypeStruct((NC, 2, NR, 128), jnp.float32),
    ],
    mesh=_mesh,
    compiler_params=_sc_params,
    scratch_types=[
        pltpu.VMEM((CF,), jnp.int32),
        pltpu.VMEM((CF,), jnp.int32),
        pltpu.VMEM((CF,), jnp.int32),
        pltpu.VMEM((CF,), jnp.int32),
        pltpu.VMEM((CF, F2), jnp.float32),
        pltpu.VMEM((CF, F2), jnp.float32),
        pltpu.VMEM((CF, F2), jnp.float32),
        pltpu.VMEM((CF, F2), jnp.float32),
        pltpu.VMEM((NR, 128), jnp.float32),
        pltpu.VMEM((NR, 128), jnp.float32),
        pltpu.VMEM((NR, 128), jnp.float32),
        pltpu.VMEM((NR, 128), jnp.float32),
        pltpu.VMEM((4, CF), jnp.float32),
        pltpu.VMEM((NR,), jnp.int32),
        pltpu.VMEM((2, L), jnp.int32),
        pltpu.VMEM_SHARED((NR, 128), jnp.float32),
        pltpu.VMEM_SHARED((NR, 128), jnp.float32),
        pltpu.SemaphoreType.DMA,
        pltpu.SemaphoreType.DMA,
        pltpu.SemaphoreType.DMA,
        pltpu.SemaphoreType.DMA,
        pltpu.SemaphoreType.DMA,
    ],
)


def _sc2_body(h, srcr, dstr, s0i, s1i, rn,
              outo,
              idxs_v, idxd_v, s0b_v, s1b_v, wb_v, rows_v, rn0_v, rn1_v, zb_v,
              out_sh):
    cid = lax.axis_index("c")
    sid = lax.axis_index("s")
    wid = sid * NC + cid
    base = wid * EW

    pltpu.sync_copy(rn.at[0], rn0_v)
    pltpu.sync_copy(rn.at[1], rn1_v)

    z16 = jnp.zeros((L,), jnp.float32)

    ZROWS = 104

    def zzb(j, carry):
        for cc in range(EMB // L):
            zb_v[j, pl.ds(cc * L, L)] = z16
        return carry

    lax.fori_loop(0, ZROWS, zzb, 0)
    myrow = sid * SROWS
    for p in range(6):
        pltpu.sync_copy(zb_v, out_sh.at[pl.ds(myrow + p * ZROWS, ZROWS)])
    pltpu.sync_copy(zb_v.at[pl.ds(0, 8)],
                    out_sh.at[pl.ds(myrow + 6 * ZROWS, 8)])
    plsc.subcore_barrier()

    C2 = 80
    NCHUNK2 = EW // C2
    NG2 = C2 // L

    def chunk(i, carry):
        off = base + i * C2
        pltpu.sync_copy(srcr.at[pl.ds(off, C2)], idxs_v)
        pltpu.sync_copy(dstr.at[pl.ds(off, C2)], idxd_v)
        pltpu.sync_copy(s0i.at[pl.ds(off, C2)], s0b_v)
        pltpu.sync_copy(s1i.at[pl.ds(off, C2)], s1b_v)
        pltpu.sync_copy(h.at[idxs_v], rows_v)

        def group(g, gcarry):
            i16s = idxs_v[pl.ds(g * L, L)]
            i16d = idxd_v[pl.ds(g * L, L)]
            rs, cs = _rc(i16s)
            rd, cd = _rc(i16d)
            w = (s0b_v[pl.ds(g * L, L)]
                 * plsc.load_gather(rn0_v, [rs, cs])
                 * plsc.load_gather(rn0_v, [rd, cd])
                 + s1b_v[pl.ds(g * L, L)]
                 * plsc.load_gather(rn1_v, [rs, cs])
                 * plsc.load_gather(rn1_v, [rd, cd]))
            wb_v[pl.ds(g * L, L)] = w
            return gcarry

        lax.fori_loop(0, NG2, group, 0)

        def edge(j, ecarry):
            wj = wb_v[pl.ds(j, L)][0]
            for cc in range(EMB // L):
                rows_v[j, pl.ds(cc * L, L)] = rows_v[j, pl.ds(cc * L, L)] * wj
            return ecarry

        lax.fori_loop(0, C2, edge, 0)

        pltpu.sync_copy(rows_v, out_sh.at[idxd_v], add=True)
        return carry

    lax.fori_loop(0, NCHUNK2, chunk, 0)

    plsc.subcore_barrier()
    pltpu.sync_copy(out_sh.at[pl.ds(myrow, SROWS)],
                    outo.at[cid, pl.ds(myrow, SROWS)])


_sc2 = pl.kernel(
    _sc2_body,
    out_type=jax.ShapeDtypeStruct((NC, NP, EMB), jnp.float32),
    mesh=_mesh,
    compiler_params=_sc_params,
    scratch_types=[
        pltpu.VMEM((80,), jnp.int32),
        pltpu.VMEM((80,), jnp.int32),
        pltpu.VMEM((80,), jnp.float32),
        pltpu.VMEM((80,), jnp.float32),
        pltpu.VMEM((80 + L,), jnp.float32),
        pltpu.VMEM((80, EMB), jnp.float32),
        pltpu.VMEM((NR, 128), jnp.float32),
        pltpu.VMEM((NR, 128), jnp.float32),
        pltpu.VMEM((104, EMB), jnp.float32),
        pltpu.VMEM_SHARED((NP, EMB), jnp.float32),
    ],
)


def _tca_body(f01_ref, wt_ref, b_ref, emb_ref, av_ref, h_ref, nmat_ref):
    x = f01_ref[...] * av_ref[...]
    h_ref[...] = (jnp.dot(x, wt_ref[...], preferred_element_type=jnp.float32)
                  + b_ref[...] + emb_ref[...])
    sq = f01_ref[...] * f01_ref[...]
    n0 = jnp.sqrt(jnp.sum(sq[:, :DM], axis=1, keepdims=True))
    n1 = jnp.sqrt(jnp.sum(sq[:, DM:], axis=1, keepdims=True))
    nmat_ref[...] = jnp.concatenate([n0, n1], axis=1)


def _tcb_body(inwp_ref, carr_ref, rn_ref):
    t = inwp_ref[0] + inwp_ref[1] + EPS
    rn_ref[...] = lax.rsqrt(t) * carr_ref[:, 0:1]


def _tcc_body(p_ref, o_ref):
    o_ref[...] = p_ref[0, :N, :] + p_ref[1, :N, :]


def kernel(f0, f1, item_emb, W, b, alphas, edge_index, k):
    alph = jax.nn.softmax(alphas, axis=0)
    a0 = alph[0]
    a1 = alph[1]

    f01 = jnp.concatenate([f0, f1], axis=1)
    src = edge_index[0]
    dst = edge_index[1]
    wt = W.T
    b2 = b.reshape(1, EMB)
    avec = jnp.concatenate([jnp.full((1, DM), a0, jnp.float32),
                            jnp.full((1, DM), a1, jnp.float32)], axis=1)

    h, nmat = pl.pallas_call(
        _tca_body,
        out_shape=[
            jax.ShapeDtypeStruct((N, EMB), jnp.float32),
            jax.ShapeDtypeStruct((N, 2), jnp.float32),
        ],
    )(f01, wt, b2, item_emb, avec)

    nrm = jnp.pad(nmat.T, ((0, 0), (0, NP - N))).reshape(2, NR, 128)

    idrows = jnp.arange(NR, dtype=jnp.int32)

    s0, s1, inwp = _sc1(f01, src, dst, nrm, idrows)

    inw2 = inwp.reshape(NC, 2, NP)
    kf = jnp.float32(k)
    carr = jnp.tile(jnp.stack([jnp.sqrt(a0 / kf), jnp.sqrt(a1 / kf)])[:, None],
                    (1, EMB))

    rn = pl.pallas_call(
        _tcb_body,
        out_shape=jax.ShapeDtypeStruct((2, NP), jnp.float32),
    )(inw2, carr)

    outp = _sc2(h, src, dst, s0, s1, rn.reshape(2, NR, 128))

    out = pl.pallas_call(
        _tcc_body,
        out_shape=jax.ShapeDtypeStruct((N, EMB), jnp.float32),
    )(outp)
    return out
